# Initial kernel scaffold; baseline (speedup 1.0000x reference)
import jax
import jax.numpy as jnp
from jax.experimental import pallas as pl


def kernel(x, rng_key, W_logits, b_logits, W_noise, b_noise):
    raise NotImplementedError("write your pallas kernel here")


# speedup vs baseline: 1.2131x; 1.2131x over previous
import functools

import jax
import jax.numpy as jnp
from jax.experimental import pallas as pl

_TOP_K = 8


def _router_body(x_ref, w_ref, b_ref, n_ref, rout_ref, idx_ref, *, bm, e, k):
    acc = jnp.dot(x_ref[...], w_ref[...],
                  preferred_element_type=jnp.float32,
                  precision=jax.lax.Precision.DEFAULT)
    acc = acc + b_ref[...]
    logits = acc[:, :e]
    nlog = acc[:, e:]
    softplus = jnp.maximum(nlog, 0.0) + jnp.log1p(jnp.exp(-jnp.abs(nlog)))
    noisy = logits + n_ref[...] * softplus

    iota_e = jax.lax.broadcasted_iota(jnp.int32, (bm, e), 1)
    iota_k = jax.lax.broadcasted_iota(jnp.int32, (bm, k), 1)
    v = noisy
    sel = jnp.zeros((bm, e), jnp.bool_)
    idx_out = jnp.zeros((bm, k), jnp.int32)
    m0 = None
    for step in range(k):
        m = jnp.max(v, axis=1, keepdims=True)
        if step == 0:
            m0 = m
        idx = jnp.min(jnp.where(v == m, iota_e, e), axis=1, keepdims=True)
        hit = iota_e == idx
        sel = jnp.logical_or(sel, hit)
        v = jnp.where(hit, -jnp.inf, v)
        idx_out = idx_out + jnp.where(iota_k == step, idx, 0)

    idx_ref[...] = idx_out
    ex = jnp.where(sel, jnp.exp(noisy - m0), 0.0)
    rout_ref[...] = ex / jnp.sum(ex, axis=1, keepdims=True)


def kernel(x, rng_key, W_logits, b_logits, W_noise, b_noise):
    b, s, d = x.shape
    e = W_logits.shape[1]
    k = _TOP_K
    m = b * s

    raw_noise = jax.random.normal(jax.random.key(rng_key), (b, s, e),
                                  dtype=jnp.float32)
    xm = x.reshape(m, d)
    nm = raw_noise.reshape(m, e)
    wc = jnp.concatenate([W_logits, W_noise], axis=1)
    bc = jnp.concatenate([b_logits, b_noise]).reshape(1, 2 * e)

    bm = 512
    grid = (m // bm,)

    rout, idx = pl.pallas_call(
        functools.partial(_router_body, bm=bm, e=e, k=k),
        grid=grid,
        in_specs=[
            pl.BlockSpec((bm, d), lambda i: (i, 0)),
            pl.BlockSpec((d, 2 * e), lambda i: (0, 0)),
            pl.BlockSpec((1, 2 * e), lambda i: (0, 0)),
            pl.BlockSpec((bm, e), lambda i: (i, 0)),
        ],
        out_specs=[
            pl.BlockSpec((bm, e), lambda i: (i, 0)),
            pl.BlockSpec((bm, k), lambda i: (i, 0)),
        ],
        out_shape=[
            jax.ShapeDtypeStruct((m, e), jnp.float32),
            jax.ShapeDtypeStruct((m, k), jnp.int32),
        ],
    )(xm, wc, bc, nm)

    return rout.reshape(b, s, e), idx.reshape(b, s, k)


# speedup vs baseline: 1.2606x; 1.0391x over previous
import functools

import jax
import jax.numpy as jnp
from jax.experimental import pallas as pl

_TOP_K = 8


def _router_body(x_ref, w_ref, b_ref, n_ref, rout_ref, idx_ref, *, bm, e, k):
    acc = jnp.dot(x_ref[...], w_ref[...],
                  preferred_element_type=jnp.float32,
                  precision=jax.lax.Precision.DEFAULT)
    acc = acc + b_ref[...]
    logits = acc[:, :e]
    nlog = acc[:, e:]
    softplus = jnp.maximum(nlog, 0.0) + jnp.log1p(jnp.exp(-jnp.abs(nlog)))
    noisy = logits + n_ref[...] * softplus

    iota_e = jax.lax.broadcasted_iota(jnp.int32, (bm, e), 1)
    iota_k = jax.lax.broadcasted_iota(jnp.int32, (bm, k), 1)
    v = noisy
    sel = jnp.zeros((bm, e), jnp.bool_)
    idx_out = jnp.zeros((bm, k), jnp.int32)
    m0 = None
    for step in range(k):
        m = jnp.max(v, axis=1, keepdims=True)
        if step == 0:
            m0 = m
        idx = jnp.min(jnp.where(v == m, iota_e, e), axis=1, keepdims=True)
        hit = iota_e == idx
        sel = jnp.logical_or(sel, hit)
        v = jnp.where(hit, -jnp.inf, v)
        idx_out = idx_out + jnp.where(iota_k == step, idx, 0)

    idx_ref[...] = idx_out
    ex = jnp.where(sel, jnp.exp(noisy - m0), 0.0)
    rout_ref[...] = ex / jnp.sum(ex, axis=1, keepdims=True)


def kernel(x, rng_key, W_logits, b_logits, W_noise, b_noise):
    b, s, d = x.shape
    e = W_logits.shape[1]
    k = _TOP_K
    m = b * s

    raw_noise = jax.random.normal(jax.random.key(rng_key), (b, s, e),
                                  dtype=jnp.float32)
    xm = x.reshape(m, d)
    nm = raw_noise.reshape(m, e)
    wc = jnp.concatenate([W_logits, W_noise], axis=1)
    bc = jnp.concatenate([b_logits, b_noise]).reshape(1, 2 * e)

    bm = 1024
    grid = (m // bm,)

    rout, idx = pl.pallas_call(
        functools.partial(_router_body, bm=bm, e=e, k=k),
        grid=grid,
        in_specs=[
            pl.BlockSpec((bm, d), lambda i: (i, 0)),
            pl.BlockSpec((d, 2 * e), lambda i: (0, 0)),
            pl.BlockSpec((1, 2 * e), lambda i: (0, 0)),
            pl.BlockSpec((bm, e), lambda i: (i, 0)),
        ],
        out_specs=[
            pl.BlockSpec((bm, e), lambda i: (i, 0)),
            pl.BlockSpec((bm, k), lambda i: (i, 0)),
        ],
        out_shape=[
            jax.ShapeDtypeStruct((m, e), jnp.float32),
            jax.ShapeDtypeStruct((m, k), jnp.int32),
        ],
    )(xm, wc, bc, nm)

    return rout.reshape(b, s, e), idx.reshape(b, s, k)
